# baseline (speedup 1.0000x reference)
import functools

import jax
import jax.numpy as jnp
from jax import lax
from jax.experimental import pallas as pl
from jax.experimental.pallas import tpu as pltpu, tpu_sc as plsc

NB = 2
C = 16384


def kernel(token_embedding, pos_table):
    B, S, E = token_embedding.shape
    NC, NS = 2, 16
    NW = NC * NS
    total = B * S * E
    per_w = total // NW
    nch = per_w // C
    pos_elems = S * E

    te_flat = token_embedding.reshape(-1)
    pos_flat = pos_table[:S].reshape(-1)

    mesh = plsc.VectorSubcoreMesh(core_axis_name="c", subcore_axis_name="s")

    @functools.partial(
        pl.kernel,
        out_type=jax.ShapeDtypeStruct((total,), jnp.float32),
        mesh=mesh,
        scratch_types=[
            pltpu.VMEM((NB, C), jnp.float32),
            pltpu.VMEM((NB, C), jnp.float32),
            pltpu.VMEM((NB, C), jnp.float32),
            pltpu.SemaphoreType.DMA((NB,)),
            pltpu.SemaphoreType.DMA((NB,)),
            pltpu.SemaphoreType.DMA((NB,)),
        ],
    )
    def sc_add(te_hbm, pos_hbm, out_hbm, te_v, pos_v, out_v, lt, lp, os):
        wid = lax.axis_index("s") * NC + lax.axis_index("c")
        base = wid * per_w

        def load_te(g, slot):
            off = pl.multiple_of(base + g * C, C)
            return pltpu.make_async_copy(
                te_hbm.at[pl.ds(off, C)], te_v.at[slot], lt.at[slot])

        def load_pos(g, slot):
            off = pl.multiple_of(base + g * C, C)
            poff = pl.multiple_of(lax.rem(off, pos_elems), C)
            return pltpu.make_async_copy(
                pos_hbm.at[pl.ds(poff, C)], pos_v.at[slot], lp.at[slot])

        def store_out(g, slot):
            off = pl.multiple_of(base + g * C, C)
            return pltpu.make_async_copy(
                out_v.at[slot], out_hbm.at[pl.ds(off, C)], os.at[slot])

        for slot in range(NB):
            load_te(slot, slot).start()
            load_pos(slot, slot).start()

        def group(gg, carry):
            for slot in range(NB):
                g = gg * NB + slot

                @pl.when(g >= NB)
                def _():
                    store_out(g - NB, slot).wait()

                load_te(g, slot).wait()
                load_pos(g, slot).wait()

                tev = te_v.at[slot]
                posv = pos_v.at[slot]
                outv = out_v.at[slot]

                @plsc.parallel_loop(0, C // 16, unroll=8)
                def _(i):
                    sl = pl.ds(i * 16, 16)
                    outv[sl] = tev[sl] + posv[sl]

                store_out(g, slot).start()

                nxt = g + NB

                @pl.when(nxt < nch)
                def _():
                    load_te(nxt, slot).start()
                    load_pos(nxt, slot).start()
            return carry

        lax.fori_loop(0, nch // NB, group, 0)

        for slot in range(NB):
            pltpu.make_async_copy(
                out_v.at[slot], out_hbm.at[pl.ds(base, C)], os.at[slot]).wait()

    out = sc_add(te_flat, pos_flat)
    return out.reshape(B, S, E)


# speedup vs baseline: 5.7016x; 5.7016x over previous
import jax
import jax.numpy as jnp
from jax.experimental import pallas as pl


def _add_kernel(te_ref, pos_ref, out_ref):
    out_ref[...] = te_ref[...] + pos_ref[...]


def kernel(token_embedding, pos_table):
    B, S, E = token_embedding.shape
    BS = 2048
    grid = (S // BS, B)
    return pl.pallas_call(
        _add_kernel,
        grid=grid,
        in_specs=[
            pl.BlockSpec((1, BS, E), lambda i, j: (j, i, 0)),
            pl.BlockSpec((BS, E), lambda i, j: (i, 0)),
        ],
        out_specs=pl.BlockSpec((1, BS, E), lambda i, j: (j, i, 0)),
        out_shape=jax.ShapeDtypeStruct((B, S, E), token_embedding.dtype),
    )(token_embedding, pos_table)
